# baseline (speedup 1.0000x reference)
import functools

import jax
import jax.numpy as jnp
from jax import lax
from jax.experimental import pallas as pl
from jax.experimental.pallas import tpu as pltpu
from jax.experimental.pallas import tpu_sc as plsc

_DIM = 32
_BLK = 128
_WAVE = 4


def _make_sc_kernel(batch):
    info = plsc.get_sparse_core_info()
    nc, ns, lanes = info.num_cores, info.num_subcores, info.num_lanes
    nw = nc * ns
    bpw = batch // nw
    nchunk = bpw // lanes

    mesh = plsc.VectorSubcoreMesh(core_axis_name="c", subcore_axis_name="s")

    @functools.partial(
        pl.kernel,
        mesh=mesh,
        compiler_params=pltpu.CompilerParams(
            needs_layout_passes=False, use_tc_tiling_on_sc=True),
        out_type=jax.ShapeDtypeStruct((batch,), jnp.float32),
        scratch_types=[
            pltpu.VMEM((bpw,), jnp.int32),
            pltpu.VMEM((bpw,), jnp.int32),
            pltpu.VMEM((_DIM, _WAVE * _BLK), jnp.float32),
            pltpu.VMEM((_DIM, _WAVE * _BLK), jnp.float32),
            pltpu.VMEM((_DIM, _WAVE * _BLK), jnp.float32),
            pltpu.VMEM((_DIM, _WAVE * _BLK), jnp.float32),
            pltpu.VMEM((bpw,), jnp.float32),
            pltpu.VMEM((lanes,), jnp.float32),
            pltpu.SemaphoreType.DMA,
            pltpu.SemaphoreType.DMA,
        ],
    )
    def k(user_hbm, item_hbm, ut_hbm, it_hbm, bias_hbm, out_hbm,
          uidx_v, iidx_v, ubuf0, ubuf1, ibuf0, ibuf1, out_v, bias_v,
          sem0, sem1):
        wid = lax.axis_index("s") * nc + lax.axis_index("c")
        base = wid * bpw

        pltpu.sync_copy(user_hbm.at[pl.ds(base, bpw)], uidx_v)
        pltpu.sync_copy(item_hbm.at[pl.ds(base, bpw)], iidx_v)
        pltpu.sync_copy(bias_hbm, bias_v)

        lane_iota = lax.iota(jnp.int32, lanes)
        d_lo = lane_iota
        d_hi = lane_iota + lanes
        bias_vec = bias_v[...]
        ubufs, ibufs, sems = (ubuf0, ubuf1), (ibuf0, ibuf1), (sem0, sem1)

        def fire(uchunk, ichunk, w4, p):
            for kk in range(_WAVE):
                ru = uchunk[w4 * _WAVE + kk]
                ri = ichunk[w4 * _WAVE + kk]
                su = pl.multiple_of((ru >> 7) * _BLK, _BLK)
                si = pl.multiple_of((ri >> 7) * _BLK, _BLK)
                dst = pl.ds(kk * _BLK, _BLK)
                pltpu.async_copy(
                    ut_hbm.at[:, pl.ds(su, _BLK)], ubufs[p].at[:, dst],
                    sems[p])
                pltpu.async_copy(
                    it_hbm.at[:, pl.ds(si, _BLK)], ibufs[p].at[:, dst],
                    sems[p])

        def wait(p):
            pltpu.make_async_copy(
                ut_hbm.at[:, pl.ds(0, _WAVE * _BLK)], ubufs[p], sems[p]
            ).wait()
            pltpu.make_async_copy(
                it_hbm.at[:, pl.ds(0, _WAVE * _BLK)], ibufs[p], sems[p]
            ).wait()

        def compute(uchunk, ichunk, w4, p, acc):
            for kk in range(_WAVE):
                pos = w4 * _WAVE + kk
                cu = (uchunk[pos] & (_BLK - 1)) + kk * _BLK
                ci = (ichunk[pos] & (_BLK - 1)) + kk * _BLK
                cuv = jnp.full((lanes,), cu, jnp.int32)
                civ = jnp.full((lanes,), ci, jnp.int32)
                u0 = plsc.load_gather(ubufs[p], [d_lo, cuv])
                u1 = plsc.load_gather(ubufs[p], [d_hi, cuv])
                i0 = plsc.load_gather(ibufs[p], [d_lo, civ])
                i1 = plsc.load_gather(ibufs[p], [d_hi, civ])
                dot = jnp.sum(u0 * i0 + u1 * i1)
                acc = jnp.where(lane_iota == pos,
                                jnp.full((lanes,), dot, jnp.float32), acc)
            return acc

        def chunk_at(j):
            return uidx_v[pl.ds(j * lanes, lanes)], iidx_v[pl.ds(j * lanes, lanes)]

        def body(j, carry):
            uchunk, ichunk = chunk_at(j)
            acc = jnp.zeros((lanes,), jnp.float32)
            fire(uchunk, ichunk, 0, 0)
            nwaves = lanes // _WAVE
            for w4 in range(nwaves):
                p = w4 & 1
                if w4 + 1 < nwaves:
                    fire(uchunk, ichunk, w4 + 1, (w4 + 1) & 1)
                wait(p)
                acc = compute(uchunk, ichunk, w4, p, acc)
            z = acc + bias_vec
            out_v[pl.ds(j * lanes, lanes)] = 1.0 / (1.0 + jnp.exp(-z))
            return carry

        lax.fori_loop(0, nchunk, body, 0)

        pltpu.sync_copy(out_v, out_hbm.at[pl.ds(base, bpw)])

    return k


def kernel(user, item, user_emb, item_emb, bias):
    batch = user.shape[0]
    lanes = plsc.get_sparse_core_info().num_lanes
    user = user.astype(jnp.int32)
    item = item.astype(jnp.int32)
    bias16 = jnp.broadcast_to(bias.astype(jnp.float32), (lanes,))
    k = _make_sc_kernel(batch)
    return k(user, item, user_emb.T, item_emb.T, bias16)
